# Initial kernel scaffold; baseline (speedup 1.0000x reference)
import jax
import jax.numpy as jnp
from jax.experimental import pallas as pl


def kernel(x, beta, edge_index):
    raise NotImplementedError("write your pallas kernel here")


# speedup vs baseline: 9.8893x; 9.8893x over previous
import functools

import jax
import jax.numpy as jnp
from jax import lax
from jax.experimental import pallas as pl
from jax.experimental.pallas import tpu as pltpu
from jax.experimental.pallas import tpu_sc as plsc

N = 10000
D = 128
E = 320000
DP = 144
NC, NS = 2, 16
NW = NC * NS
EPT = E // NW
C = 80
NCHUNK = EPT // C
RPS = N // NS
DUMP = N + 8


def _post_body(u_ref, d_ref, o_ref):
    usum = u_ref[0] + u_ref[1]
    dsum = d_ref[0] + d_ref[1]
    inv = jnp.where(dsum > 0, 1.0 / dsum, 0.0)
    o_ref[...] = usum * inv


def _sc_edge(x_hbm, ei_hbm, beta_hbm, u_hbm, d_hbm,
             ridx_v, cidx_v, sidx_v, a_v, b_v, wb_v, w_v, beta_v, z1_v,
             u_sh, d_sh, semA, semB):
    c = lax.axis_index("c")
    s = lax.axis_index("s")
    ebase = (c * NS + s) * EPT
    zero16 = jnp.zeros((16,), jnp.float32)
    lane = jnp.arange(16, dtype=jnp.int32)

    def zrow(i, _):
        for j in range(8):
            wb_v[i, pl.ds(j * 16, 16)] = zero16
        return 0
    lax.fori_loop(0, C, zrow, 0)
    def z1(j, _):
        z1_v[pl.ds(j * 16, 16)] = zero16
        return 0
    lax.fori_loop(0, 62, z1, 0)
    z1_v[pl.ds(984, 16)] = zero16

    @pl.when(s < 10)
    def _():
        def zcp(k, _):
            pltpu.sync_copy(wb_v.at[pl.ds(0, 40)],
                            u_sh.at[pl.ds(s * 1000 + k * 40, 40)])
            return 0
        lax.fori_loop(0, 25, zcp, 0)
        pltpu.sync_copy(z1_v, d_sh.at[pl.ds(s * 1000, 1000)])
    plsc.subcore_barrier()

    pltpu.sync_copy(beta_hbm, beta_v.at[pl.ds(0, 1)])
    bvec = jnp.full((16,), beta_v[pl.ds(0, 16)][0])

    def chunk(k, _):
        base = ebase + k * C
        pltpu.sync_copy(ei_hbm.at[pl.ds(base, C)], ridx_v.at[pl.ds(16, C)])
        pltpu.sync_copy(ei_hbm.at[pl.ds(E + base, C)], cidx_v)
        cpa = pltpu.async_copy(x_hbm.at[ridx_v.at[pl.ds(16, C)]], a_v, semA)
        cpb = pltpu.async_copy(x_hbm.at[cidx_v], b_v, semB)
        cpa.wait()
        cpb.wait()

        def grp(g, _):
            e0 = g * 16

            dots = zero16
            ssa = zero16
            ssb = zero16
            for t in range(16):
                e = e0 + t
                dacc, aacc, bacc = zero16, zero16, zero16
                for j in range(8):
                    va = a_v[e, pl.ds(j * 16, 16)]
                    vb = b_v[e, pl.ds(j * 16, 16)]
                    dacc = dacc + va * vb
                    aacc = aacc + va * va
                    bacc = bacc + vb * vb
                hot = lane == t
                dots = dots + jnp.where(hot, jnp.full((16,), jnp.sum(dacc)), 0.0)
                ssa = ssa + jnp.where(hot, jnp.full((16,), jnp.sum(aacc)), 0.0)
                ssb = ssb + jnp.where(hot, jnp.full((16,), jnp.sum(bacc)), 0.0)

            v = ssa * ssb
            seed = (plsc.bitcast(v, jnp.int32) >> 1) + 0x1FBD1DF5
            y = plsc.bitcast(seed, jnp.float32)
            y = 0.5 * (y + v / y)
            y = 0.5 * (y + v / y)
            y = 0.5 * (y + v / y)
            w16 = jnp.exp(bvec * dots / (y + 1e-7))
            w_v[pl.ds(e0, 16)] = w16

            for t in range(16):
                e = e0 + t
                wv = jnp.full((16,), w16[t])
                for j in range(8):
                    wb_v[e, pl.ds(j * 16, 16)] = b_v[e, pl.ds(j * 16, 16)] * wv

            r16 = ridx_v[pl.ds(16 + e0, 16)]
            shift1 = jnp.maximum(lane - 1, 0)
            occ = jnp.zeros((16,), jnp.int32)
            sh = r16
            for dl in range(1, 16):
                sh = sh.at[shift1].get(mode="promise_in_bounds")
                occ = occ + jnp.where((lane >= dl) & (r16 == sh), 1, 0)

            for r in range(16):
                sel = occ == r
                nsel = plsc.all_reduce_population_count(sel)
                @pl.when(nsel[0] > 0)
                def _():
                    sidx_v[...] = jnp.where(sel, r16, DUMP)
                    pltpu.sync_copy(wb_v.at[pl.ds(e0, 16)],
                                    u_sh.at[sidx_v], add=True)
                    pltpu.sync_copy(w_v.at[pl.ds(e0, 16)],
                                    d_sh.at[sidx_v], add=True)
            return 0
        lax.fori_loop(0, C // 16, grp, 0)
        return 0
    lax.fori_loop(0, NCHUNK, chunk, 0)

    plsc.subcore_barrier()
    @pl.when(s < 15)
    def _():
        pltpu.sync_copy(u_sh.at[pl.ds(s * 624, 624)],
                        u_hbm.at[c, pl.ds(s * 624, 624)])
    @pl.when(s == 15)
    def _():
        pltpu.sync_copy(u_sh.at[pl.ds(9360, 640)],
                        u_hbm.at[c, pl.ds(9360, 640)])
    @pl.when(s < 10)
    def _():
        pltpu.sync_copy(d_sh.at[pl.ds(s * 1000, 1000)], z1_v)
        pltpu.sync_copy(z1_v, d_hbm.at[pl.ds(c * N + s * 1000, 1000)])


_sc_call = functools.partial(
    pl.kernel,
    out_type=[
        jax.ShapeDtypeStruct((NC, N, D), jnp.float32),
        jax.ShapeDtypeStruct((NC * N,), jnp.float32),
    ],
    mesh=plsc.VectorSubcoreMesh(
        core_axis_name="c", subcore_axis_name="s",
        num_cores=NC, num_subcores=NS),
    compiler_params=pltpu.CompilerParams(needs_layout_passes=False),
    scratch_types=[
        pltpu.VMEM((C + 16,), jnp.int32),
        pltpu.VMEM((C,), jnp.int32),
        pltpu.VMEM((16,), jnp.int32),
        pltpu.VMEM((C, D), jnp.float32),
        pltpu.VMEM((C, D), jnp.float32),
        pltpu.VMEM((C, D), jnp.float32),
        pltpu.VMEM((C,), jnp.float32),
        pltpu.VMEM((16,), jnp.float32),
        pltpu.VMEM((1000,), jnp.float32),
        pltpu.VMEM_SHARED((N + 16, D), jnp.float32),
        pltpu.VMEM_SHARED((N + 16,), jnp.float32),
        pltpu.SemaphoreType.DMA,
        pltpu.SemaphoreType.DMA,
    ],
)(_sc_edge)


@jax.jit
def kernel(x, beta, edge_index):
    ei = edge_index.astype(jnp.int32).reshape(-1)
    u, d = _sc_call(x, ei, beta)
    out = pl.pallas_call(
        _post_body,
        out_shape=jax.ShapeDtypeStruct((N, D), jnp.float32),
    )(u, d.reshape(NC, N, 1))
    return out


# speedup vs baseline: 10.9480x; 1.1071x over previous
import functools

import jax
import jax.numpy as jnp
from jax import lax
from jax.experimental import pallas as pl
from jax.experimental.pallas import tpu as pltpu
from jax.experimental.pallas import tpu_sc as plsc

N = 10000
D = 128
E = 320000
DP = 144
NC, NS = 2, 16
NW = NC * NS
EPT = E // NW
C = 80
NCHUNK = EPT // C
RPS = N // NS
DUMP = N + 8


def _post_body(u_ref, d_ref, o_ref):
    usum = u_ref[0] + u_ref[1]
    dsum = d_ref[0] + d_ref[1]
    inv = jnp.where(dsum > 0, 1.0 / dsum, 0.0)
    o_ref[...] = usum * inv


def _sc_edge(x_hbm, ei_hbm, beta_hbm, u_hbm, d_hbm,
             ridx_v, cidx_v, sidx_v, a_v, b_v, wb_v, w_v, beta_v, z1_v,
             u_sh, d_sh, semA, semB, semU, semD):
    c = lax.axis_index("c")
    s = lax.axis_index("s")
    ebase = (c * NS + s) * EPT
    zero16 = jnp.zeros((16,), jnp.float32)
    lane = jnp.arange(16, dtype=jnp.int32)

    def zrow(i, _):
        for j in range(8):
            wb_v[i, pl.ds(j * 16, 16)] = zero16
        return 0
    lax.fori_loop(0, C, zrow, 0)
    def z1(j, _):
        z1_v[pl.ds(j * 16, 16)] = zero16
        return 0
    lax.fori_loop(0, 62, z1, 0)
    z1_v[pl.ds(984, 16)] = zero16

    @pl.when(s < 10)
    def _():
        def zcp(k, _):
            pltpu.sync_copy(wb_v.at[pl.ds(0, 40)],
                            u_sh.at[pl.ds(s * 1000 + k * 40, 40)])
            return 0
        lax.fori_loop(0, 25, zcp, 0)
        pltpu.sync_copy(z1_v, d_sh.at[pl.ds(s * 1000, 1000)])
    plsc.subcore_barrier()

    pltpu.sync_copy(beta_hbm, beta_v.at[pl.ds(0, 1)])
    bvec = jnp.full((16,), beta_v[pl.ds(0, 16)][0])

    def chunk(k, _):
        base = ebase + k * C
        pltpu.sync_copy(ei_hbm.at[pl.ds(base, C)], ridx_v.at[pl.ds(16, C)])
        pltpu.sync_copy(ei_hbm.at[pl.ds(E + base, C)], cidx_v)
        cpa = pltpu.async_copy(x_hbm.at[ridx_v.at[pl.ds(16, C)]], a_v, semA)
        cpb = pltpu.async_copy(x_hbm.at[cidx_v], b_v, semB)
        cpa.wait()
        cpb.wait()

        def grp(g, _):
            e0 = g * 16

            dots = zero16
            ssa = zero16
            ssb = zero16
            for t in range(16):
                e = e0 + t
                dacc, aacc, bacc = zero16, zero16, zero16
                for j in range(8):
                    va = a_v[e, pl.ds(j * 16, 16)]
                    vb = b_v[e, pl.ds(j * 16, 16)]
                    dacc = dacc + va * vb
                    aacc = aacc + va * va
                    bacc = bacc + vb * vb
                hot = lane == t
                dots = dots + jnp.where(hot, jnp.full((16,), jnp.sum(dacc)), 0.0)
                ssa = ssa + jnp.where(hot, jnp.full((16,), jnp.sum(aacc)), 0.0)
                ssb = ssb + jnp.where(hot, jnp.full((16,), jnp.sum(bacc)), 0.0)

            v = ssa * ssb
            seed = (plsc.bitcast(v, jnp.int32) >> 1) + 0x1FBD1DF5
            y = plsc.bitcast(seed, jnp.float32)
            y = 0.5 * (y + v / y)
            y = 0.5 * (y + v / y)
            y = 0.5 * (y + v / y)
            w16 = jnp.exp(bvec * dots / (y + 1e-7))
            w_v[pl.ds(e0, 16)] = w16

            for t in range(16):
                e = e0 + t
                wv = jnp.full((16,), w16[t])
                for j in range(8):
                    wb_v[e, pl.ds(j * 16, 16)] = b_v[e, pl.ds(j * 16, 16)] * wv

            r16 = ridx_v[pl.ds(16 + e0, 16)]
            shift1 = jnp.maximum(lane - 1, 0)
            occ = jnp.zeros((16,), jnp.int32)
            sh = r16
            for dl in range(1, 16):
                sh = sh.at[shift1].get(mode="promise_in_bounds")
                occ = occ + jnp.where((lane >= dl) & (r16 == sh), 1, 0)

            sidx_v[g, pl.ds(0, 16)] = jnp.where(occ == 0, r16, DUMP)
            for r in range(1, 16):
                sel = occ == r
                nsel = plsc.all_reduce_population_count(sel)
                @pl.when(nsel[0] > 0)
                def _():
                    sidx_v[8 + g, pl.ds(0, 16)] = jnp.where(sel, r16, DUMP)
                    pltpu.sync_copy(wb_v.at[pl.ds(e0, 16)],
                                    u_sh.at[sidx_v.at[8 + g]], add=True)
                    pltpu.sync_copy(w_v.at[pl.ds(e0, 16)],
                                    d_sh.at[sidx_v.at[8 + g]], add=True)
            return 0
        lax.fori_loop(0, C // 16, grp, 0)

        cps = []
        for g in range(C // 16):
            cps.append(pltpu.async_copy(
                wb_v.at[pl.ds(g * 16, 16)], u_sh.at[sidx_v.at[g]],
                semU, add=True))
            cps.append(pltpu.async_copy(
                w_v.at[pl.ds(g * 16, 16)], d_sh.at[sidx_v.at[g]],
                semD, add=True))
        for cp in cps:
            cp.wait()
        return 0
    lax.fori_loop(0, NCHUNK, chunk, 0)

    plsc.subcore_barrier()
    @pl.when(s < 15)
    def _():
        pltpu.sync_copy(u_sh.at[pl.ds(s * 624, 624)],
                        u_hbm.at[c, pl.ds(s * 624, 624)])
    @pl.when(s == 15)
    def _():
        pltpu.sync_copy(u_sh.at[pl.ds(9360, 640)],
                        u_hbm.at[c, pl.ds(9360, 640)])
    @pl.when(s < 10)
    def _():
        pltpu.sync_copy(d_sh.at[pl.ds(s * 1000, 1000)], z1_v)
        pltpu.sync_copy(z1_v, d_hbm.at[pl.ds(c * N + s * 1000, 1000)])


_sc_call = functools.partial(
    pl.kernel,
    out_type=[
        jax.ShapeDtypeStruct((NC, N, D), jnp.float32),
        jax.ShapeDtypeStruct((NC * N,), jnp.float32),
    ],
    mesh=plsc.VectorSubcoreMesh(
        core_axis_name="c", subcore_axis_name="s",
        num_cores=NC, num_subcores=NS),
    compiler_params=pltpu.CompilerParams(needs_layout_passes=False),
    scratch_types=[
        pltpu.VMEM((C + 16,), jnp.int32),
        pltpu.VMEM((C,), jnp.int32),
        pltpu.VMEM((16, 16), jnp.int32),
        pltpu.VMEM((C, D), jnp.float32),
        pltpu.VMEM((C, D), jnp.float32),
        pltpu.VMEM((C, D), jnp.float32),
        pltpu.VMEM((C,), jnp.float32),
        pltpu.VMEM((16,), jnp.float32),
        pltpu.VMEM((1000,), jnp.float32),
        pltpu.VMEM_SHARED((N + 16, D), jnp.float32),
        pltpu.VMEM_SHARED((N + 16,), jnp.float32),
        pltpu.SemaphoreType.DMA,
        pltpu.SemaphoreType.DMA,
        pltpu.SemaphoreType.DMA,
        pltpu.SemaphoreType.DMA,
    ],
)(_sc_edge)


@jax.jit
def kernel(x, beta, edge_index):
    ei = edge_index.astype(jnp.int32).reshape(-1)
    u, d = _sc_call(x, ei, beta)
    out = pl.pallas_call(
        _post_body,
        out_shape=jax.ShapeDtypeStruct((N, D), jnp.float32),
    )(u, d.reshape(NC, N, 1))
    return out


# speedup vs baseline: 13.7509x; 1.2560x over previous
import functools

import jax
import jax.numpy as jnp
from jax import lax
from jax.experimental import pallas as pl
from jax.experimental.pallas import tpu as pltpu
from jax.experimental.pallas import tpu_sc as plsc

N = 10000
D = 128
E = 320000
DP = 144
NC, NS = 2, 16
NW = NC * NS
EPT = E // NW
C = 80
NCHUNK = EPT // C
RPS = N // NS
DUMP = N + 8


def _post_body(u_ref, d_ref, o_ref):
    usum = u_ref[0] + u_ref[1]
    dsum = d_ref[0] + d_ref[1]
    inv = jnp.where(dsum > 0, 1.0 / dsum, 0.0)
    o_ref[...] = usum * inv


def _sc_edge(x_hbm, ei_hbm, beta_hbm, u_hbm, d_hbm,
             ridx0, cidx0, ridx1, cidx1, sidx_v,
             a0, b0, a1, b1, w_v, beta_v, z1_v,
             u_sh, d_sh, semG0, semG1, semU, semD):
    c = lax.axis_index("c")
    s = lax.axis_index("s")
    ebase = (c * NS + s) * EPT
    zero16 = jnp.zeros((16,), jnp.float32)
    lane = jnp.arange(16, dtype=jnp.int32)
    RIDX, CIDX = (ridx0, ridx1), (cidx0, cidx1)
    AV, BV, SEMG = (a0, a1), (b0, b1), (semG0, semG1)

    def zrow(i, _):
        for j in range(8):
            b0[i, pl.ds(j * 16, 16)] = zero16
        return 0
    lax.fori_loop(0, C, zrow, 0)
    def z1(j, _):
        z1_v[pl.ds(j * 16, 16)] = zero16
        return 0
    lax.fori_loop(0, 62, z1, 0)
    z1_v[pl.ds(984, 16)] = zero16

    @pl.when(s < 10)
    def _():
        def zcp(k, _):
            pltpu.sync_copy(b0.at[pl.ds(0, 40)],
                            u_sh.at[pl.ds(s * 1000 + k * 40, 40)])
            return 0
        lax.fori_loop(0, 25, zcp, 0)
        pltpu.sync_copy(z1_v, d_sh.at[pl.ds(s * 1000, 1000)])
    plsc.subcore_barrier()

    pltpu.sync_copy(beta_hbm, beta_v.at[pl.ds(0, 1)])
    bvec = jnp.full((16,), beta_v[pl.ds(0, 16)][0])

    def issue(ck, p):
        base = ebase + ck * C
        pltpu.sync_copy(ei_hbm.at[pl.ds(base, C)], RIDX[p].at[pl.ds(16, C)])
        pltpu.sync_copy(ei_hbm.at[pl.ds(E + base, C)], CIDX[p])
        pltpu.async_copy(x_hbm.at[RIDX[p].at[pl.ds(16, C)]], AV[p], SEMG[p])
        pltpu.async_copy(x_hbm.at[CIDX[p]], BV[p], SEMG[p])

    def process(p, next_ck):
        a_v, b_v, ridx_v = AV[p], BV[p], RIDX[p]
        pltpu.make_async_copy(
            x_hbm.at[ridx_v.at[pl.ds(16, C)]], a_v, SEMG[p]).wait()
        pltpu.make_async_copy(x_hbm.at[CIDX[p]], b_v, SEMG[p]).wait()
        if next_ck is not None:
            issue(next_ck, 1 - p)

        def grp(g, _):
            e0 = g * 16

            dots = zero16
            ssa = zero16
            ssb = zero16
            for t in range(16):
                e = e0 + t
                dacc, aacc, bacc = zero16, zero16, zero16
                for j in range(8):
                    va = a_v[e, pl.ds(j * 16, 16)]
                    vb = b_v[e, pl.ds(j * 16, 16)]
                    dacc = dacc + va * vb
                    aacc = aacc + va * va
                    bacc = bacc + vb * vb
                hot = lane == t
                dots = dots + jnp.where(hot, jnp.full((16,), jnp.sum(dacc)), 0.0)
                ssa = ssa + jnp.where(hot, jnp.full((16,), jnp.sum(aacc)), 0.0)
                ssb = ssb + jnp.where(hot, jnp.full((16,), jnp.sum(bacc)), 0.0)

            v = ssa * ssb
            seed = (plsc.bitcast(v, jnp.int32) >> 1) + 0x1FBD1DF5
            y = plsc.bitcast(seed, jnp.float32)
            y = 0.5 * (y + v / y)
            y = 0.5 * (y + v / y)
            y = 0.5 * (y + v / y)
            w16 = jnp.exp(bvec * dots / (y + 1e-7))
            w_v[pl.ds(e0, 16)] = w16

            for t in range(16):
                e = e0 + t
                wv = jnp.full((16,), w16[t])
                for j in range(8):
                    b_v[e, pl.ds(j * 16, 16)] = b_v[e, pl.ds(j * 16, 16)] * wv

            r16 = ridx_v[pl.ds(16 + e0, 16)]
            shift1 = jnp.maximum(lane - 1, 0)
            occ = jnp.zeros((16,), jnp.int32)
            sh = r16
            for dl in range(1, 16):
                sh = sh.at[shift1].get(mode="promise_in_bounds")
                occ = occ + jnp.where((lane >= dl) & (r16 == sh), 1, 0)

            sidx_v[g, pl.ds(0, 16)] = jnp.where(occ == 0, r16, DUMP)
            for r in range(1, 16):
                sel = occ == r
                nsel = plsc.all_reduce_population_count(sel)
                @pl.when(nsel[0] > 0)
                def _():
                    sidx_v[8 + g, pl.ds(0, 16)] = jnp.where(sel, r16, DUMP)
                    pltpu.sync_copy(b_v.at[pl.ds(e0, 16)],
                                    u_sh.at[sidx_v.at[8 + g]], add=True)
                    pltpu.sync_copy(w_v.at[pl.ds(e0, 16)],
                                    d_sh.at[sidx_v.at[8 + g]], add=True)
            return 0
        lax.fori_loop(0, C // 16, grp, 0)

        cps = []
        for g in range(C // 16):
            cps.append(pltpu.async_copy(
                b_v.at[pl.ds(g * 16, 16)], u_sh.at[sidx_v.at[g]],
                semU, add=True))
            cps.append(pltpu.async_copy(
                w_v.at[pl.ds(g * 16, 16)], d_sh.at[sidx_v.at[g]],
                semD, add=True))
        for cp in cps:
            cp.wait()

    issue(0, 0)
    def pipe(j, _):
        process(0, 2 * j + 1)
        process(1, 2 * j + 2)
        return 0
    lax.fori_loop(0, (NCHUNK - 1) // 2, pipe, 0)
    process(0, None)

    plsc.subcore_barrier()
    @pl.when(s < 15)
    def _():
        pltpu.sync_copy(u_sh.at[pl.ds(s * 624, 624)],
                        u_hbm.at[c, pl.ds(s * 624, 624)])
    @pl.when(s == 15)
    def _():
        pltpu.sync_copy(u_sh.at[pl.ds(9360, 640)],
                        u_hbm.at[c, pl.ds(9360, 640)])
    @pl.when(s < 10)
    def _():
        pltpu.sync_copy(d_sh.at[pl.ds(s * 1000, 1000)], z1_v)
        pltpu.sync_copy(z1_v, d_hbm.at[pl.ds(c * N + s * 1000, 1000)])


_sc_call = functools.partial(
    pl.kernel,
    out_type=[
        jax.ShapeDtypeStruct((NC, N, D), jnp.float32),
        jax.ShapeDtypeStruct((NC * N,), jnp.float32),
    ],
    mesh=plsc.VectorSubcoreMesh(
        core_axis_name="c", subcore_axis_name="s",
        num_cores=NC, num_subcores=NS),
    compiler_params=pltpu.CompilerParams(needs_layout_passes=False),
    scratch_types=[
        pltpu.VMEM((C + 16,), jnp.int32),
        pltpu.VMEM((C,), jnp.int32),
        pltpu.VMEM((C + 16,), jnp.int32),
        pltpu.VMEM((C,), jnp.int32),
        pltpu.VMEM((16, 16), jnp.int32),
        pltpu.VMEM((C, D), jnp.float32),
        pltpu.VMEM((C, D), jnp.float32),
        pltpu.VMEM((C, D), jnp.float32),
        pltpu.VMEM((C, D), jnp.float32),
        pltpu.VMEM((C,), jnp.float32),
        pltpu.VMEM((16,), jnp.float32),
        pltpu.VMEM((1000,), jnp.float32),
        pltpu.VMEM_SHARED((N + 16, D), jnp.float32),
        pltpu.VMEM_SHARED((N + 16,), jnp.float32),
        pltpu.SemaphoreType.DMA,
        pltpu.SemaphoreType.DMA,
        pltpu.SemaphoreType.DMA,
        pltpu.SemaphoreType.DMA,
    ],
)(_sc_edge)


@jax.jit
def kernel(x, beta, edge_index):
    ei = edge_index.astype(jnp.int32).reshape(-1)
    u, d = _sc_call(x, ei, beta)
    out = pl.pallas_call(
        _post_body,
        out_shape=jax.ShapeDtypeStruct((N, D), jnp.float32),
    )(u, d.reshape(NC, N, 1))
    return out


# speedup vs baseline: 13.7963x; 1.0033x over previous
import functools

import jax
import jax.numpy as jnp
from jax import lax
from jax.experimental import pallas as pl
from jax.experimental.pallas import tpu as pltpu
from jax.experimental.pallas import tpu_sc as plsc

N = 10000
D = 128
E = 320000
DP = 144
NC, NS = 2, 16
NW = NC * NS
EPT = E // NW
C = 80
NCHUNK = EPT // C
RPS = N // NS
DUMP = N + 8


def _post_body(u_ref, d_ref, o_ref):
    usum = u_ref[0] + u_ref[1]
    dsum = d_ref[0] + d_ref[1]
    inv = jnp.where(dsum > 0, 1.0 / dsum, 0.0)
    o_ref[...] = usum * inv


def _sc_edge(x_hbm, ei_hbm, beta_hbm, u_hbm, d_hbm,
             ridx0, cidx0, ridx1, cidx1, sidx_v,
             a0, b0, a1, b1, w_v, beta_v, z1_v,
             u_sh, d_sh, semG0, semG1, semU, semD):
    c = lax.axis_index("c")
    s = lax.axis_index("s")
    ebase = (c * NS + s) * EPT
    zero16 = jnp.zeros((16,), jnp.float32)
    lane = jnp.arange(16, dtype=jnp.int32)
    RIDX, CIDX = (ridx0, ridx1), (cidx0, cidx1)
    AV, BV, SEMG = (a0, a1), (b0, b1), (semG0, semG1)

    def zrow(i, _):
        for j in range(8):
            b0[i, pl.ds(j * 16, 16)] = zero16
        return 0
    lax.fori_loop(0, C, zrow, 0)
    def z1(j, _):
        z1_v[pl.ds(j * 16, 16)] = zero16
        return 0
    lax.fori_loop(0, 62, z1, 0)
    z1_v[pl.ds(984, 16)] = zero16

    @pl.when(s < 10)
    def _():
        def zcp(k, _):
            pltpu.sync_copy(b0.at[pl.ds(0, 40)],
                            u_sh.at[pl.ds(s * 1000 + k * 40, 40)])
            return 0
        lax.fori_loop(0, 25, zcp, 0)
        pltpu.sync_copy(z1_v, d_sh.at[pl.ds(s * 1000, 1000)])
    plsc.subcore_barrier()

    pltpu.sync_copy(beta_hbm, beta_v.at[pl.ds(0, 1)])
    bvec = jnp.full((16,), beta_v[pl.ds(0, 16)][0])

    def issue(ck, p):
        base = ebase + ck * C
        pltpu.sync_copy(ei_hbm.at[pl.ds(base, C)], RIDX[p].at[pl.ds(16, C)])
        pltpu.sync_copy(ei_hbm.at[pl.ds(E + base, C)], CIDX[p])
        pltpu.async_copy(x_hbm.at[RIDX[p].at[pl.ds(16, C)]], AV[p], SEMG[p])
        pltpu.async_copy(x_hbm.at[CIDX[p]], BV[p], SEMG[p])

    def drain(p):
        b_v = BV[p]
        for g in range(C // 16):
            pltpu.make_async_copy(
                b_v.at[pl.ds(g * 16, 16)], u_sh.at[sidx_v.at[g]], semU).wait()
            pltpu.make_async_copy(
                w_v.at[pl.ds(g * 16, 16)], d_sh.at[sidx_v.at[g]], semD).wait()

    def process(p, next_ck, drain_guard=None):
        a_v, b_v, ridx_v = AV[p], BV[p], RIDX[p]
        pltpu.make_async_copy(
            x_hbm.at[ridx_v.at[pl.ds(16, C)]], a_v, SEMG[p]).wait()
        pltpu.make_async_copy(x_hbm.at[CIDX[p]], b_v, SEMG[p]).wait()
        if drain_guard is True:
            drain(1 - p)
        elif drain_guard is not None:
            pl.when(drain_guard)(lambda: drain(1 - p))
        if next_ck is not None:
            issue(next_ck, 1 - p)

        def grp(g, _):
            e0 = g * 16

            dots = zero16
            ssa = zero16
            ssb = zero16
            for t in range(16):
                e = e0 + t
                dacc, aacc, bacc = zero16, zero16, zero16
                for j in range(8):
                    va = a_v[e, pl.ds(j * 16, 16)]
                    vb = b_v[e, pl.ds(j * 16, 16)]
                    dacc = dacc + va * vb
                    aacc = aacc + va * va
                    bacc = bacc + vb * vb
                hot = lane == t
                dots = dots + jnp.where(hot, jnp.full((16,), jnp.sum(dacc)), 0.0)
                ssa = ssa + jnp.where(hot, jnp.full((16,), jnp.sum(aacc)), 0.0)
                ssb = ssb + jnp.where(hot, jnp.full((16,), jnp.sum(bacc)), 0.0)

            v = ssa * ssb
            seed = (plsc.bitcast(v, jnp.int32) >> 1) + 0x1FBD1DF5
            y = plsc.bitcast(seed, jnp.float32)
            y = 0.5 * (y + v / y)
            y = 0.5 * (y + v / y)
            y = 0.5 * (y + v / y)
            w16 = jnp.exp(bvec * dots / (y + 1e-7))
            w_v[pl.ds(e0, 16)] = w16

            for t in range(16):
                e = e0 + t
                wv = jnp.full((16,), w16[t])
                for j in range(8):
                    b_v[e, pl.ds(j * 16, 16)] = b_v[e, pl.ds(j * 16, 16)] * wv

            r16 = ridx_v[pl.ds(16 + e0, 16)]
            shift1 = jnp.maximum(lane - 1, 0)
            occ = jnp.zeros((16,), jnp.int32)
            sh = r16
            for dl in range(1, 16):
                sh = sh.at[shift1].get(mode="promise_in_bounds")
                occ = occ + jnp.where((lane >= dl) & (r16 == sh), 1, 0)

            sidx_v[g, pl.ds(0, 16)] = jnp.where(occ == 0, r16, DUMP)
            for r in range(1, 16):
                sel = occ == r
                nsel = plsc.all_reduce_population_count(sel)
                @pl.when(nsel[0] > 0)
                def _():
                    sidx_v[8 + g, pl.ds(0, 16)] = jnp.where(sel, r16, DUMP)
                    pltpu.sync_copy(b_v.at[pl.ds(e0, 16)],
                                    u_sh.at[sidx_v.at[8 + g]], add=True)
                    pltpu.sync_copy(w_v.at[pl.ds(e0, 16)],
                                    d_sh.at[sidx_v.at[8 + g]], add=True)
            return 0
        lax.fori_loop(0, C // 16, grp, 0)

        cps = []
        for g in range(C // 16):
            cps.append(pltpu.async_copy(
                b_v.at[pl.ds(g * 16, 16)], u_sh.at[sidx_v.at[g]],
                semU, add=True))
            cps.append(pltpu.async_copy(
                w_v.at[pl.ds(g * 16, 16)], d_sh.at[sidx_v.at[g]],
                semD, add=True))

    issue(0, 0)
    def pipe(j, _):
        process(0, 2 * j + 1, drain_guard=j > 0)
        process(1, 2 * j + 2, drain_guard=True)
        return 0
    lax.fori_loop(0, (NCHUNK - 1) // 2, pipe, 0)
    process(0, None, drain_guard=True)
    drain(0)

    plsc.subcore_barrier()
    @pl.when(s < 15)
    def _():
        pltpu.sync_copy(u_sh.at[pl.ds(s * 624, 624)],
                        u_hbm.at[c, pl.ds(s * 624, 624)])
    @pl.when(s == 15)
    def _():
        pltpu.sync_copy(u_sh.at[pl.ds(9360, 640)],
                        u_hbm.at[c, pl.ds(9360, 640)])
    @pl.when(s < 10)
    def _():
        pltpu.sync_copy(d_sh.at[pl.ds(s * 1000, 1000)], z1_v)
        pltpu.sync_copy(z1_v, d_hbm.at[pl.ds(c * N + s * 1000, 1000)])


_sc_call = functools.partial(
    pl.kernel,
    out_type=[
        jax.ShapeDtypeStruct((NC, N, D), jnp.float32),
        jax.ShapeDtypeStruct((NC * N,), jnp.float32),
    ],
    mesh=plsc.VectorSubcoreMesh(
        core_axis_name="c", subcore_axis_name="s",
        num_cores=NC, num_subcores=NS),
    compiler_params=pltpu.CompilerParams(needs_layout_passes=False),
    scratch_types=[
        pltpu.VMEM((C + 16,), jnp.int32),
        pltpu.VMEM((C,), jnp.int32),
        pltpu.VMEM((C + 16,), jnp.int32),
        pltpu.VMEM((C,), jnp.int32),
        pltpu.VMEM((16, 16), jnp.int32),
        pltpu.VMEM((C, D), jnp.float32),
        pltpu.VMEM((C, D), jnp.float32),
        pltpu.VMEM((C, D), jnp.float32),
        pltpu.VMEM((C, D), jnp.float32),
        pltpu.VMEM((C,), jnp.float32),
        pltpu.VMEM((16,), jnp.float32),
        pltpu.VMEM((1000,), jnp.float32),
        pltpu.VMEM_SHARED((N + 16, D), jnp.float32),
        pltpu.VMEM_SHARED((N + 16,), jnp.float32),
        pltpu.SemaphoreType.DMA,
        pltpu.SemaphoreType.DMA,
        pltpu.SemaphoreType.DMA,
        pltpu.SemaphoreType.DMA,
    ],
)(_sc_edge)


@jax.jit
def kernel(x, beta, edge_index):
    ei = edge_index.astype(jnp.int32).reshape(-1)
    u, d = _sc_call(x, ei, beta)
    out = pl.pallas_call(
        _post_body,
        out_shape=jax.ShapeDtypeStruct((N, D), jnp.float32),
    )(u, d.reshape(NC, N, 1))
    return out
